# baseline (speedup 1.0000x reference)
import functools

import jax
import jax.numpy as jnp
from jax import lax
from jax.experimental import pallas as pl
from jax.experimental.pallas import tpu as pltpu
from jax.experimental.pallas import tpu_sc as plsc

N_ROWS = 4096
N_COLS = 200
B = N_ROWS * N_COLS
D = 128
V = 45
NC = 2
NS = 16
NW = NC * NS
BPW = B // NW
C = 128
CHUNK_W = C * D
NCHUNK = BPW // C
L = 16


def _embed_body(table_hbm, idx_hbm, out_hbm, table_v, idx_v, rows0, rows1,
                ss0, ss1):
    wid = lax.axis_index("s") * NC + lax.axis_index("c")
    base = wid * BPW

    pltpu.sync_copy(table_hbm, table_v)
    pltpu.sync_copy(idx_hbm.at[wid], idx_v)

    lane = lax.broadcasted_iota(jnp.int32, (L,), 0)
    out_lane = lane * D

    def build_chunk(i, buf):
        def rowgrp(g, _):
            idx16 = idx_v[pl.ds(i * C + g * L, L)]
            base_in = idx16 * D

            def col(cc, _):
                a_in = base_in + cc
                vals = plsc.load_gather(table_v, [a_in])
                a_out = out_lane + (g * (L * D) + cc)
                plsc.store_scatter(buf, [a_out], vals)
                return 0

            lax.fori_loop(0, D, col, 0, unroll=8)
            return 0

        lax.fori_loop(0, C // L, rowgrp, 0)

    def store(i, buf, sem):
        return pltpu.async_copy(
            buf, out_hbm.at[pl.ds((base + i * C) * D, CHUNK_W)], sem)

    def wait_store(i, buf, sem):
        pltpu.make_async_copy(
            buf, out_hbm.at[pl.ds((base + i * C) * D, CHUNK_W)], sem).wait()

    build_chunk(0, rows0)
    store(0, rows0, ss0)

    def body(j, carry):
        a = 2 * j + 1

        @pl.when(j > 0)
        def _():
            wait_store(a - 2, rows1, ss1)

        build_chunk(a, rows1)
        store(a, rows1, ss1)

        @pl.when(j < NCHUNK // 2 - 1)
        def _():
            wait_store(a - 1, rows0, ss0)
            build_chunk(a + 1, rows0)
            store(a + 1, rows0, ss0)

        return carry

    lax.fori_loop(0, NCHUNK // 2, body, 0)
    wait_store(NCHUNK - 2, rows0, ss0)
    wait_store(NCHUNK - 1, rows1, ss1)


def kernel(inputs, table):
    idx = inputs.reshape(NW, BPW).astype(jnp.int32)
    tflat = table.reshape(V * D)
    mesh = plsc.VectorSubcoreMesh(core_axis_name="c", subcore_axis_name="s")
    out = pl.kernel(
        _embed_body,
        mesh=mesh,
        out_type=jax.ShapeDtypeStruct((B * D,), jnp.float32),
        compiler_params=pltpu.CompilerParams(needs_layout_passes=False),
        scratch_types=[
            pltpu.VMEM((V * D,), jnp.float32),
            pltpu.VMEM((BPW,), jnp.int32),
            pltpu.VMEM((CHUNK_W,), jnp.float32),
            pltpu.VMEM((CHUNK_W,), jnp.float32),
            pltpu.SemaphoreType.DMA,
            pltpu.SemaphoreType.DMA,
        ],
    )(tflat, idx)
    return out.reshape(N_ROWS, N_COLS, D)


# speedup vs baseline: 10.0861x; 10.0861x over previous
import functools

import jax
import jax.numpy as jnp
from jax import lax
from jax.experimental import pallas as pl
from jax.experimental.pallas import tpu as pltpu
from jax.experimental.pallas import tpu_sc as plsc

N_ROWS = 4096
N_COLS = 200
B = N_ROWS * N_COLS
D = 128
V = 45
NC = 2
NS = 16
NW = NC * NS
BPW = B // NW
C = 128
CHUNK_W = C * D
NCHUNK = BPW // C
L = 16


def _embed_body(table_hbm, idx_hbm, out_hbm, table_v, idx_v, rows0, rows1,
                ss0, ss1):
    wid = lax.axis_index("s") * NC + lax.axis_index("c")
    base = wid * BPW

    pltpu.sync_copy(table_hbm, table_v)
    pltpu.sync_copy(idx_hbm.at[wid], idx_v)

    def build_chunk(i, buf):
        @plsc.parallel_loop(0, C // L)
        def _grp(g):
            idx16 = idx_v[pl.ds(i * C + g * L, L)]
            for l in range(L):
                src = idx16[l] * D
                dst = g * (L * D) + l * D
                for c in range(D // L):
                    buf[pl.ds(dst + c * L, L)] = table_v[pl.ds(src + c * L, L)]

    def store(i, buf, sem):
        return pltpu.async_copy(
            buf, out_hbm.at[pl.ds((base + i * C) * D, CHUNK_W)], sem)

    def wait_store(i, buf, sem):
        pltpu.make_async_copy(
            buf, out_hbm.at[pl.ds((base + i * C) * D, CHUNK_W)], sem).wait()

    build_chunk(0, rows0)
    store(0, rows0, ss0)

    def body(j, carry):
        a = 2 * j + 1

        @pl.when(j > 0)
        def _():
            wait_store(a - 2, rows1, ss1)

        build_chunk(a, rows1)
        store(a, rows1, ss1)

        @pl.when(j < NCHUNK // 2 - 1)
        def _():
            wait_store(a - 1, rows0, ss0)
            build_chunk(a + 1, rows0)
            store(a + 1, rows0, ss0)

        return carry

    lax.fori_loop(0, NCHUNK // 2, body, 0)
    wait_store(NCHUNK - 2, rows0, ss0)
    wait_store(NCHUNK - 1, rows1, ss1)


def kernel(inputs, table):
    idx = inputs.reshape(NW, BPW).astype(jnp.int32)
    tflat = table.reshape(V * D)
    mesh = plsc.VectorSubcoreMesh(core_axis_name="c", subcore_axis_name="s")
    out = pl.kernel(
        _embed_body,
        mesh=mesh,
        out_type=jax.ShapeDtypeStruct((B * D,), jnp.float32),
        compiler_params=pltpu.CompilerParams(needs_layout_passes=False),
        scratch_types=[
            pltpu.VMEM((V * D,), jnp.float32),
            pltpu.VMEM((BPW,), jnp.int32),
            pltpu.VMEM((CHUNK_W,), jnp.float32),
            pltpu.VMEM((CHUNK_W,), jnp.float32),
            pltpu.SemaphoreType.DMA,
            pltpu.SemaphoreType.DMA,
        ],
    )(tflat, idx)
    return out.reshape(N_ROWS, N_COLS, D)


# speedup vs baseline: 18.8949x; 1.8734x over previous
import functools

import jax
import jax.numpy as jnp
from jax import lax
from jax.experimental import pallas as pl
from jax.experimental.pallas import tpu as pltpu
from jax.experimental.pallas import tpu_sc as plsc

N_ROWS = 4096
N_COLS = 200
B = N_ROWS * N_COLS
D = 128
V = 45
NC = 2
NS = 16
NW = NC * NS
BPW = B // NW
C = 128
NCHUNK = BPW // C


def _embed_body(table_hbm, idx_hbm, out_hbm, table_s, idx_v, rows0, rows1,
                sg0, sg1, ss0, ss1):
    wid = lax.axis_index("s") * NC + lax.axis_index("c")
    base = wid * BPW

    @pl.when(lax.axis_index("s") == 0)
    def _():
        pltpu.sync_copy(table_hbm, table_s)

    pltpu.sync_copy(idx_hbm.at[wid], idx_v)
    plsc.subcore_barrier()

    def gather(i, buf, sem):
        return pltpu.async_copy(table_s.at[idx_v.at[i]], buf, sem)

    def store(i, buf, sem):
        return pltpu.async_copy(buf, out_hbm.at[pl.ds(base + i * C, C)], sem)

    def wait_gather(i, buf, sem):
        pltpu.make_async_copy(table_s.at[idx_v.at[i]], buf, sem).wait()

    def wait_store(i, buf, sem):
        pltpu.make_async_copy(
            buf, out_hbm.at[pl.ds(base + i * C, C)], sem).wait()

    gather(0, rows0, sg0)

    def body(j, carry):
        a = 2 * j
        b = a + 1
        wait_gather(a, rows0, sg0)
        store(a, rows0, ss0)

        @pl.when(j > 0)
        def _():
            wait_store(b - 2, rows1, ss1)

        gather(b, rows1, sg1)
        wait_gather(b, rows1, sg1)
        store(b, rows1, ss1)
        wait_store(a, rows0, ss0)

        @pl.when(j < NCHUNK // 2 - 1)
        def _():
            gather(a + 2, rows0, sg0)

        return carry

    lax.fori_loop(0, NCHUNK // 2, body, 0)
    wait_store(NCHUNK - 1, rows1, ss1)


def kernel(inputs, table):
    idx = inputs.reshape(NW, NCHUNK, C).astype(jnp.int32)
    mesh = plsc.VectorSubcoreMesh(core_axis_name="c", subcore_axis_name="s")
    out = pl.kernel(
        _embed_body,
        mesh=mesh,
        out_type=jax.ShapeDtypeStruct((B, D), jnp.float32),
        compiler_params=pltpu.CompilerParams(needs_layout_passes=False),
        scratch_types=[
            pltpu.VMEM_SHARED((V, D), jnp.float32),
            pltpu.VMEM((NCHUNK, C), jnp.int32),
            pltpu.VMEM((C, D), jnp.float32),
            pltpu.VMEM((C, D), jnp.float32),
            pltpu.SemaphoreType.DMA,
            pltpu.SemaphoreType.DMA,
            pltpu.SemaphoreType.DMA,
            pltpu.SemaphoreType.DMA,
        ],
    )(table, idx)
    return out.reshape(N_ROWS, N_COLS, D)


# speedup vs baseline: 20.2227x; 1.0703x over previous
import functools

import jax
import jax.numpy as jnp
from jax import lax
from jax.experimental import pallas as pl
from jax.experimental.pallas import tpu as pltpu
from jax.experimental.pallas import tpu_sc as plsc

N_ROWS = 4096
N_COLS = 200
B = N_ROWS * N_COLS
D = 128
V = 45
NC = 2
NS = 16
NW = NC * NS
BPW = B // NW
C = 128
NCHUNK = BPW // C
NB = 4
NJ = NCHUNK // NB


def _embed_body(table_hbm, idx_hbm, out_hbm, table_s, idx_v, bufs, sg, ss):
    wid = lax.axis_index("s") * NC + lax.axis_index("c")
    base = wid * BPW

    @pl.when(lax.axis_index("s") == 0)
    def _():
        pltpu.sync_copy(table_hbm, table_s)

    pltpu.sync_copy(idx_hbm.at[wid], idx_v)
    plsc.subcore_barrier()

    def gather(i, k):
        pltpu.async_copy(table_s.at[idx_v.at[i]], bufs[k], sg[k])

    def store(i, k):
        pltpu.async_copy(bufs[k], out_hbm.at[pl.ds(base + i * C, C)], ss[k])

    def wait_gather(i, k):
        pltpu.make_async_copy(table_s.at[idx_v.at[i]], bufs[k], sg[k]).wait()

    def wait_store(i, k):
        pltpu.make_async_copy(
            bufs[k], out_hbm.at[pl.ds(base + i * C, C)], ss[k]).wait()

    gather(0, 0)
    gather(1, 1)

    def body(j, carry):
        for k in range(NB):
            i = NB * j + k
            wait_gather(i, k)
            store(i, k)
            k2 = (k + 2) % NB
            if k < 2:
                @pl.when(j > 0)
                def _():
                    wait_store(i - 2, k2)

                gather(i + 2, k2)
            else:
                @pl.when(j < NJ - 1)
                def _():
                    wait_store(i - 2, k2)
                    gather(i + 2, k2)
        return carry

    lax.fori_loop(0, NJ, body, 0)
    for k in range(NB):
        wait_store(NCHUNK - NB + k, k)


def kernel(inputs, table):
    idx = inputs.reshape(NW, NCHUNK, C).astype(jnp.int32)
    mesh = plsc.VectorSubcoreMesh(core_axis_name="c", subcore_axis_name="s")
    out = pl.kernel(
        _embed_body,
        mesh=mesh,
        out_type=jax.ShapeDtypeStruct((B, D), jnp.float32),
        compiler_params=pltpu.CompilerParams(needs_layout_passes=False),
        scratch_types=[
            pltpu.VMEM_SHARED((V, D), jnp.float32),
            pltpu.VMEM((NCHUNK, C), jnp.int32),
            [pltpu.VMEM((C, D), jnp.float32) for _ in range(NB)],
            [pltpu.SemaphoreType.DMA for _ in range(NB)],
            [pltpu.SemaphoreType.DMA for _ in range(NB)],
        ],
    )(table, idx)
    return out.reshape(N_ROWS, N_COLS, D)


# speedup vs baseline: 20.3926x; 1.0084x over previous
import functools

import jax
import jax.numpy as jnp
from jax import lax
from jax.experimental import pallas as pl
from jax.experimental.pallas import tpu as pltpu
from jax.experimental.pallas import tpu_sc as plsc

N_ROWS = 4096
N_COLS = 200
B = N_ROWS * N_COLS
D = 128
V = 45
NC = 2
NS = 16
NW = NC * NS
BPW = B // NW
C = 128
NCHUNK = BPW // C
NB = 5
A = 3
NJ = NCHUNK // NB


def _embed_body(table_hbm, idx_hbm, out_hbm, table_s, idx_v, bufs, sg, ss):
    wid = lax.axis_index("s") * NC + lax.axis_index("c")
    base = wid * BPW

    @pl.when(lax.axis_index("s") == 0)
    def _():
        pltpu.sync_copy(table_hbm, table_s)

    pltpu.sync_copy(idx_hbm.at[wid], idx_v)
    plsc.subcore_barrier()

    def gather(i, k):
        pltpu.async_copy(table_s.at[idx_v.at[i]], bufs[k], sg[k])

    def store(i, k):
        pltpu.async_copy(bufs[k], out_hbm.at[pl.ds(base + i * C, C)], ss[k])

    def wait_gather(i, k):
        pltpu.make_async_copy(table_s.at[idx_v.at[i]], bufs[k], sg[k]).wait()

    def wait_store(i, k):
        pltpu.make_async_copy(
            bufs[k], out_hbm.at[pl.ds(base + i * C, C)], ss[k]).wait()

    for i in range(A):
        gather(i, i)

    def body(j, carry):
        for k in range(NB):
            i = NB * j + k
            wait_gather(i, k)
            store(i, k)
            kA = (k + A) % NB
            if k < NB - A:
                @pl.when(j > 0)
                def _():
                    wait_store(i + A - NB, kA)

                gather(i + A, kA)
            else:
                @pl.when(j < NJ - 1)
                def _():
                    wait_store(i + A - NB, kA)
                    gather(i + A, kA)
        return carry

    lax.fori_loop(0, NJ, body, 0)
    for k in range(NB):
        wait_store(NCHUNK - NB + k, k)


def kernel(inputs, table):
    idx = inputs.reshape(NW, NCHUNK, C).astype(jnp.int32)
    mesh = plsc.VectorSubcoreMesh(core_axis_name="c", subcore_axis_name="s")
    out = pl.kernel(
        _embed_body,
        mesh=mesh,
        out_type=jax.ShapeDtypeStruct((B, D), jnp.float32),
        compiler_params=pltpu.CompilerParams(needs_layout_passes=False),
        scratch_types=[
            pltpu.VMEM_SHARED((V, D), jnp.float32),
            pltpu.VMEM((NCHUNK, C), jnp.int32),
            [pltpu.VMEM((C, D), jnp.float32) for _ in range(NB)],
            [pltpu.SemaphoreType.DMA for _ in range(NB)],
            [pltpu.SemaphoreType.DMA for _ in range(NB)],
        ],
    )(table, idx)
    return out.reshape(N_ROWS, N_COLS, D)
